# Initial kernel scaffold; baseline (speedup 1.0000x reference)
import functools

import jax
import jax.numpy as jnp
from jax import lax
from jax.experimental import pallas as pl
from jax.experimental.pallas import tpu as pltpu
from jax.experimental.pallas import tpu_sc as plsc

DIM = 1024
D_FF = 1024
E = 8
TOPK = 2
S = 2048
TM = 128
NPAD = S * TOPK + E * TM
NB = NPAD // TM
NW = 32


def _router_body(x_ref, wg_ref, bg_ref, mi_ref, mf_ref, ob_ref, run_ref):
    b = pl.program_id(0)
    lane = lax.broadcasted_iota(jnp.int32, (256, 128), 1)

    @pl.when(b < 8)
    def _route():
        run = jnp.where(b == 0, 0.0, run_ref[0:1, :])
        logits = jnp.dot(x_ref[...], wg_ref[...],
                         preferred_element_type=jnp.float32) + bg_ref[0:1, :]
        masked = jnp.where(lane < E, logits, -1e30)
        v0 = jnp.max(masked, axis=-1, keepdims=True)
        e0 = jnp.min(jnp.where(masked == v0, lane, 999), axis=-1, keepdims=True)
        l1 = jnp.where(lane == e0, -1e30, masked)
        v1 = jnp.max(l1, axis=-1, keepdims=True)
        e1 = jnp.min(jnp.where(l1 == v1, lane, 999), axis=-1, keepdims=True)
        s = jnp.exp(v1 - v0)
        w0 = 1.0 / (1.0 + s)
        w1 = s / (1.0 + s)
        oh0 = (lane == e0).astype(jnp.float32)
        oh1 = (lane == e1).astype(jnp.float32)
        cnt = oh0 + oh1
        ir = lax.broadcasted_iota(jnp.int32, (256, 256), 0)
        ic = lax.broadcasted_iota(jnp.int32, (256, 256), 1)
        tri = (ir > ic).astype(jnp.float32)
        excl = jnp.dot(tri, cnt, preferred_element_type=jnp.float32) + run
        r0 = jnp.sum(excl * oh0, axis=-1, keepdims=True).astype(jnp.int32)
        r1 = jnp.sum(excl * oh1, axis=-1, keepdims=True).astype(jnp.int32)
        mi_ref[...] = jnp.where(
            lane == 0, e0, jnp.where(lane == 1, e1,
            jnp.where(lane == 2, r0, jnp.where(lane == 3, r1, 0))))
        mf_ref[...] = jnp.where(lane == 0, w0, jnp.where(lane == 1, w1, 0.0))
        run_ref[0:1, :] = run + jnp.sum(cnt, axis=0, keepdims=True)

    @pl.when(b == 8)
    def _finalize():
        lane1 = lane[0:1, :]
        tot = run_ref[0:1, :].astype(jnp.int32)
        nb = (tot + (TM - 1)) // TM
        off_vec = jnp.zeros((1, 128), jnp.int32)
        be_cnt = jnp.zeros((1, 128), jnp.int32)
        incl = jnp.int32(0)
        for e in range(E):
            nb_e = jnp.sum(jnp.where(lane1 == e, nb, 0))
            off_vec = off_vec + (incl * TM) * (lane1 == e).astype(jnp.int32)
            incl = incl + nb_e
            be_cnt = be_cnt + ((lane1 - E) >= incl).astype(jnp.int32)
        be_vec = jnp.minimum(be_cnt, E - 1)
        packed = jnp.where(lane1 < E, off_vec, be_vec)
        ob_ref[...] = jnp.broadcast_to(packed, (8, 128))


def _run_router(xs, wgp, bgp):
    return pl.pallas_call(
        _router_body,
        grid=(9,),
        in_specs=[
            pl.BlockSpec((256, DIM), lambda b: (jnp.minimum(b, 7), 0)),
            pl.BlockSpec((DIM, 128), lambda b: (0, 0)),
            pl.BlockSpec((8, 128), lambda b: (0, 0)),
        ],
        out_specs=[
            pl.BlockSpec((256, 128), lambda b: (jnp.minimum(b, 7), 0)),
            pl.BlockSpec((256, 128), lambda b: (jnp.minimum(b, 7), 0)),
            pl.BlockSpec((8, 128), lambda b: (0, 0)),
        ],
        out_shape=[
            jax.ShapeDtypeStruct((S, 128), jnp.int32),
            jax.ShapeDtypeStruct((S, 128), jnp.float32),
            jax.ShapeDtypeStruct((8, 128), jnp.int32),
        ],
        scratch_shapes=[pltpu.VMEM((8, 128), jnp.float32)],
    )(xs, wgp, bgp)


def _make_dispatch():
    mesh = plsc.VectorSubcoreMesh(core_axis_name="c", subcore_axis_name="s")

    @functools.partial(
        pl.kernel, mesh=mesh,
        out_type=[
            jax.ShapeDtypeStruct((NPAD,), jnp.int32),
            jax.ShapeDtypeStruct((NPAD,), jnp.float32),
            jax.ShapeDtypeStruct((S * TOPK,), jnp.int32),
        ],
        scratch_types=[
            pltpu.VMEM((S * TOPK,), jnp.int32),
            pltpu.VMEM((S * TOPK,), jnp.int32),
            pltpu.VMEM((S * TOPK,), jnp.float32),
            pltpu.VMEM((16,), jnp.int32),
            pltpu.VMEM((NPAD,), jnp.int32),
            pltpu.VMEM((NPAD,), jnp.float32),
            pltpu.VMEM((S * TOPK,), jnp.int32),
        ],
    )
    def dispatch(e_hbm, r_hbm, w_hbm, off_hbm, tok_out, wrow_out, dest_out,
                 ev, rv, wv, offv, btok, bw, bdest):
        wid = lax.axis_index("s") * 2 + lax.axis_index("c")

        @pl.when(wid == 0)
        def _work():
            pltpu.sync_copy(e_hbm, ev)
            pltpu.sync_copy(r_hbm, rv)
            pltpu.sync_copy(w_hbm, wv)
            pltpu.sync_copy(off_hbm, offv)
            zi = jnp.zeros((16,), jnp.int32)
            zf = jnp.zeros((16,), jnp.float32)

            def _zero(i, _):
                sl = pl.ds(pl.multiple_of(i * 16, 16), 16)
                btok[sl] = zi
                bw[sl] = zf
                return 0
            lax.fori_loop(0, NPAD // 16, _zero, 0)

            iota = lax.broadcasted_iota(jnp.int32, (16,), 0)

            def _scatter(i, _):
                sl = pl.ds(pl.multiple_of(i * 16, 16), 16)
                e16 = ev[sl]
                offp = plsc.load_gather(offv, [e16])
                d16 = offp + rv[sl]
                tok = (i * 16 + iota) >> 1
                plsc.store_scatter(btok, [d16], tok)
                plsc.store_scatter(bw, [d16], wv[sl])
                bdest[sl] = d16
                return 0
            lax.fori_loop(0, (S * TOPK) // 16, _scatter, 0)

            pltpu.sync_copy(btok, tok_out)
            pltpu.sync_copy(bw, wrow_out)
            pltpu.sync_copy(bdest, dest_out)

    return dispatch


_dispatch = _make_dispatch()


def _make_gather():
    mesh = plsc.VectorSubcoreMesh(core_axis_name="c", subcore_axis_name="s")
    bpw = NPAD // NW

    @functools.partial(
        pl.kernel, mesh=mesh,
        out_type=jax.ShapeDtypeStruct((NPAD, DIM), jnp.float32),
        scratch_types=[
            pltpu.VMEM((bpw,), jnp.int32),
            pltpu.VMEM((16, DIM), jnp.float32),
            pltpu.SemaphoreType.DMA,
        ],
    )
    def gather(tok_hbm, x_hbm, xg_out, idxv, rows, sem):
        wid = lax.axis_index("s") * 2 + lax.axis_index("c")
        base = pl.multiple_of(wid * bpw, 8)
        pltpu.sync_copy(tok_hbm.at[pl.ds(base, bpw)], idxv)

        def _chunk(c, _):
            i16 = idxv[pl.ds(pl.multiple_of(c * 16, 16), 16)]
            pltpu.async_copy(x_hbm.at[i16], rows, sem).wait()
            pltpu.sync_copy(rows, xg_out.at[pl.ds(base + c * 16, 16)])
            return 0
        lax.fori_loop(0, bpw // 16, _chunk, 0)

    return gather


_gather = _make_gather()


def _ffn_body(be_ref, xg_ref, wr_ref, w1_ref, b1_ref, w2_ref, b2_ref, o_ref):
    xb = xg_ref[...].astype(jnp.bfloat16)
    h = jnp.dot(xb, w1_ref[0], preferred_element_type=jnp.float32) + b1_ref[...]
    a1 = h[:, :D_FF]
    a2 = h[:, D_FF:]
    act = a1 * jax.nn.sigmoid(a1) * a2
    o = jnp.dot(act.astype(jnp.bfloat16), w2_ref[0],
                preferred_element_type=jnp.float32) + b2_ref[...]
    o_ref[...] = o * wr_ref[...]


def _run_ffn(be, xg, wrow, w1b, b1, w2b, b2):
    grid_spec = pltpu.PrefetchScalarGridSpec(
        num_scalar_prefetch=1,
        grid=(NB,),
        in_specs=[
            pl.BlockSpec((TM, DIM), lambda b, be: (b, 0)),
            pl.BlockSpec((TM, 1), lambda b, be: (b, 0)),
            pl.BlockSpec((1, DIM, 2 * D_FF), lambda b, be: (be[b], 0, 0)),
            pl.BlockSpec((1, 2 * D_FF), lambda b, be: (be[b], 0)),
            pl.BlockSpec((1, D_FF, DIM), lambda b, be: (be[b], 0, 0)),
            pl.BlockSpec((1, DIM), lambda b, be: (be[b], 0)),
        ],
        out_specs=pl.BlockSpec((TM, DIM), lambda b, be: (b, 0)),
    )
    return pl.pallas_call(
        _ffn_body,
        grid_spec=grid_spec,
        out_shape=jax.ShapeDtypeStruct((NPAD, DIM), jnp.float32),
    )(be, xg, wrow, w1b, b1, w2b, b2)


def _make_combine():
    mesh = plsc.VectorSubcoreMesh(core_axis_name="c", subcore_axis_name="s")
    tpw = S // NW

    @functools.partial(
        pl.kernel, mesh=mesh,
        out_type=jax.ShapeDtypeStruct((S, DIM), jnp.float32),
        scratch_types=[
            pltpu.VMEM((tpw,), jnp.int32),
            pltpu.VMEM((tpw,), jnp.int32),
            pltpu.VMEM((16, DIM), jnp.float32),
            pltpu.VMEM((16, DIM), jnp.float32),
            pltpu.SemaphoreType.DMA,
            pltpu.SemaphoreType.DMA,
        ],
    )
    def combine(d0_hbm, d1_hbm, o_hbm, y_out, d0v, d1v, r0, r1, sem0, sem1):
        wid = lax.axis_index("s") * 2 + lax.axis_index("c")
        base = pl.multiple_of(wid * tpw, 8)
        pltpu.sync_copy(d0_hbm.at[pl.ds(base, tpw)], d0v)
        pltpu.sync_copy(d1_hbm.at[pl.ds(base, tpw)], d1v)

        def _chunk(c, _):
            i0 = d0v[pl.ds(pl.multiple_of(c * 16, 16), 16)]
            i1 = d1v[pl.ds(pl.multiple_of(c * 16, 16), 16)]
            cp0 = pltpu.async_copy(o_hbm.at[i0], r0, sem0)
            cp1 = pltpu.async_copy(o_hbm.at[i1], r1, sem1)
            cp0.wait()
            cp1.wait()

            def _add(l, _):
                sl = pl.ds(pl.multiple_of(l * 16, 16), 16)
                for j in range(16):
                    r0[j, sl] = r0[j, sl] + r1[j, sl]
                return 0
            lax.fori_loop(0, DIM // 16, _add, 0)
            pltpu.sync_copy(r0, y_out.at[pl.ds(base + c * 16, 16)])
            return 0
        lax.fori_loop(0, tpw // 16, _chunk, 0)

    return combine


_combine = _make_combine()


def kernel(x, Wg, bg, W1, b1, W2, b2):
    xs = x.reshape(S, DIM)
    wgp = jnp.pad(Wg, ((0, 0), (0, 128 - E)))
    bgp = jnp.broadcast_to(jnp.pad(bg, (0, 128 - E))[None, :], (8, 128))

    meta_i, meta_f, off_be = _run_router(xs, wgp, bgp)

    e_flat = meta_i[:, 0:2].reshape(-1)
    r_flat = meta_i[:, 2:4].reshape(-1)
    w_flat = meta_f[:, 0:2].reshape(-1)
    off16 = jnp.pad(off_be[0, :E], (0, 16 - E))

    src_tok, wrow, dest = _dispatch(e_flat, r_flat, w_flat, off16)
    xg = _gather(src_tok, xs)

    be = off_be[0, E:E + NB]
    o_big = _run_ffn(be, xg, wrow.reshape(NPAD, 1),
                     W1.astype(jnp.bfloat16), b1,
                     W2.astype(jnp.bfloat16), b2)

    d2 = dest.reshape(S, TOPK)
    y = _combine(d2[:, 0], d2[:, 1], o_big)
    return y.reshape(x.shape)


# speedup vs baseline: 1.1366x; 1.1366x over previous
import functools

import jax
import jax.numpy as jnp
from jax import lax
from jax.experimental import pallas as pl
from jax.experimental.pallas import tpu as pltpu
from jax.experimental.pallas import tpu_sc as plsc

DIM = 1024
D_FF = 1024
E = 8
TOPK = 2
S = 2048
TM = 128
NPAD = S * TOPK + E * TM
NB = NPAD // TM
NW = 32


def _router_body(x_ref, wg_ref, bg_ref, mi_ref, mf_ref, ob_ref, run_ref):
    b = pl.program_id(0)
    lane = lax.broadcasted_iota(jnp.int32, (256, 128), 1)

    @pl.when(b < 8)
    def _route():
        run = jnp.where(b == 0, 0.0, run_ref[0:1, :])
        logits = jnp.dot(x_ref[...], wg_ref[...],
                         preferred_element_type=jnp.float32) + bg_ref[0:1, :]
        masked = jnp.where(lane < E, logits, -1e30)
        v0 = jnp.max(masked, axis=-1, keepdims=True)
        e0 = jnp.min(jnp.where(masked == v0, lane, 999), axis=-1, keepdims=True)
        l1 = jnp.where(lane == e0, -1e30, masked)
        v1 = jnp.max(l1, axis=-1, keepdims=True)
        e1 = jnp.min(jnp.where(l1 == v1, lane, 999), axis=-1, keepdims=True)
        s = jnp.exp(v1 - v0)
        w0 = 1.0 / (1.0 + s)
        w1 = s / (1.0 + s)
        oh0 = (lane == e0).astype(jnp.float32)
        oh1 = (lane == e1).astype(jnp.float32)
        cnt = oh0 + oh1
        ir = lax.broadcasted_iota(jnp.int32, (256, 256), 0)
        ic = lax.broadcasted_iota(jnp.int32, (256, 256), 1)
        tri = (ir > ic).astype(jnp.float32)
        excl = jnp.dot(tri, cnt, preferred_element_type=jnp.float32) + run
        r0 = jnp.sum(excl * oh0, axis=-1, keepdims=True).astype(jnp.int32)
        r1 = jnp.sum(excl * oh1, axis=-1, keepdims=True).astype(jnp.int32)
        mi_ref[...] = jnp.where(
            lane == 0, e0, jnp.where(lane == 1, e1,
            jnp.where(lane == 2, r0, jnp.where(lane == 3, r1, 0))))
        mf_ref[...] = jnp.where(lane == 0, w0, jnp.where(lane == 1, w1, 0.0))
        run_ref[0:1, :] = run + jnp.sum(cnt, axis=0, keepdims=True)

    @pl.when(b == 8)
    def _finalize():
        lane1 = lane[0:1, :]
        tot = run_ref[0:1, :].astype(jnp.int32)
        nb = (tot + (TM - 1)) // TM
        off_vec = jnp.zeros((1, 128), jnp.int32)
        be_cnt = jnp.zeros((1, 128), jnp.int32)
        incl = jnp.int32(0)
        for e in range(E):
            nb_e = jnp.sum(jnp.where(lane1 == e, nb, 0))
            off_vec = off_vec + (incl * TM) * (lane1 == e).astype(jnp.int32)
            incl = incl + nb_e
            be_cnt = be_cnt + ((lane1 - E) >= incl).astype(jnp.int32)
        be_vec = jnp.minimum(be_cnt, E - 1)
        packed = jnp.where(lane1 < E, off_vec, be_vec)
        ob_ref[...] = jnp.broadcast_to(packed, (8, 128))


def _run_router(xs, wgp, bgp):
    return pl.pallas_call(
        _router_body,
        grid=(9,),
        in_specs=[
            pl.BlockSpec((256, DIM), lambda b: (jnp.minimum(b, 7), 0)),
            pl.BlockSpec((DIM, 128), lambda b: (0, 0)),
            pl.BlockSpec((8, 128), lambda b: (0, 0)),
        ],
        out_specs=[
            pl.BlockSpec((256, 128), lambda b: (jnp.minimum(b, 7), 0)),
            pl.BlockSpec((256, 128), lambda b: (jnp.minimum(b, 7), 0)),
            pl.BlockSpec((8, 128), lambda b: (0, 0)),
        ],
        out_shape=[
            jax.ShapeDtypeStruct((S, 128), jnp.int32),
            jax.ShapeDtypeStruct((S, 128), jnp.float32),
            jax.ShapeDtypeStruct((8, 128), jnp.int32),
        ],
        scratch_shapes=[pltpu.VMEM((8, 128), jnp.float32)],
    )(xs, wgp, bgp)


def _dest_body(mi_ref, ob_ref, d_ref):
    lane = lax.broadcasted_iota(jnp.int32, (S, 128), 1)
    lane1 = lane[0:1, :]
    off_row = ob_ref[0:1, :]
    e0 = mi_ref[:, 0:1]
    e1 = mi_ref[:, 1:2]
    d0 = mi_ref[:, 2:3]
    d1 = mi_ref[:, 3:4]
    for e in range(E):
        off_e = jnp.sum(jnp.where(lane1 == e, off_row, 0))
        d0 = d0 + off_e * (e0 == e).astype(jnp.int32)
        d1 = d1 + off_e * (e1 == e).astype(jnp.int32)
    d_ref[...] = jnp.where(lane == 0, d0, jnp.where(lane == 1, d1, 0))


def _run_dest(meta_i, off_be):
    return pl.pallas_call(
        _dest_body,
        out_shape=jax.ShapeDtypeStruct((S, 128), jnp.int32),
    )(meta_i, off_be)


def _make_dispatch():
    mesh = plsc.VectorSubcoreMesh(core_axis_name="c", subcore_axis_name="s")
    tpw = S // NW

    @functools.partial(
        pl.kernel, mesh=mesh,
        out_type=jax.ShapeDtypeStruct((NPAD, DIM), jnp.float32),
        scratch_types=[
            pltpu.VMEM((tpw,), jnp.int32),
            pltpu.VMEM((tpw,), jnp.int32),
            pltpu.VMEM((tpw, DIM), jnp.float32),
            pltpu.SemaphoreType.DMA,
        ],
    )
    def dispatch(d0_hbm, d1_hbm, x_hbm, xg_out, d0v, d1v, rows, sem):
        wid = lax.axis_index("s") * 2 + lax.axis_index("c")
        base = pl.multiple_of(wid * tpw, 8)
        pltpu.sync_copy(d0_hbm.at[pl.ds(base, tpw)], d0v)
        pltpu.sync_copy(d1_hbm.at[pl.ds(base, tpw)], d1v)
        pltpu.sync_copy(x_hbm.at[pl.ds(base, tpw)], rows)
        cp0 = pltpu.async_copy(rows, xg_out.at[d0v], sem)
        cp1 = pltpu.async_copy(rows, xg_out.at[d1v], sem)
        cp0.wait()
        cp1.wait()

    return dispatch


_make_dispatch = functools.cache(_make_dispatch)


def _ffn_body(be_ref, xg_ref, w1_ref, b1_ref, w2_ref, b2_ref, o_ref):
    xb = xg_ref[...].astype(jnp.bfloat16)
    h = jnp.dot(xb, w1_ref[0], preferred_element_type=jnp.float32) + b1_ref[0]
    a1 = h[:, :D_FF]
    a2 = h[:, D_FF:]
    act = a1 * jax.nn.sigmoid(a1) * a2
    o = jnp.dot(act.astype(jnp.bfloat16), w2_ref[0],
                preferred_element_type=jnp.float32) + b2_ref[0]
    o_ref[...] = o


def _run_ffn(be, xg, w1b, b1, w2b, b2):
    grid_spec = pltpu.PrefetchScalarGridSpec(
        num_scalar_prefetch=1,
        grid=(NB,),
        in_specs=[
            pl.BlockSpec((TM, DIM), lambda b, be: (b, 0)),
            pl.BlockSpec((1, DIM, 2 * D_FF), lambda b, be: (be[b], 0, 0)),
            pl.BlockSpec((1, 1, 2 * D_FF), lambda b, be: (be[b], 0, 0)),
            pl.BlockSpec((1, D_FF, DIM), lambda b, be: (be[b], 0, 0)),
            pl.BlockSpec((1, 1, DIM), lambda b, be: (be[b], 0, 0)),
        ],
        out_specs=pl.BlockSpec((TM, DIM), lambda b, be: (b, 0)),
    )
    return pl.pallas_call(
        _ffn_body,
        grid_spec=grid_spec,
        out_shape=jax.ShapeDtypeStruct((NPAD, DIM), jnp.float32),
    )(be, xg, w1b, b1, w2b, b2)


def _make_collect():
    mesh = plsc.VectorSubcoreMesh(core_axis_name="c", subcore_axis_name="s")
    tpw = S // NW

    @functools.partial(
        pl.kernel, mesh=mesh,
        out_type=[
            jax.ShapeDtypeStruct((S, DIM), jnp.float32),
            jax.ShapeDtypeStruct((S, DIM), jnp.float32),
        ],
        scratch_types=[
            pltpu.VMEM((tpw,), jnp.int32),
            pltpu.VMEM((tpw,), jnp.int32),
            pltpu.VMEM((tpw, DIM), jnp.float32),
            pltpu.SemaphoreType.DMA,
        ],
    )
    def collect(d0_hbm, d1_hbm, o_hbm, a_out, b_out, d0v, d1v, rows, sem):
        wid = lax.axis_index("s") * 2 + lax.axis_index("c")
        base = pl.multiple_of(wid * tpw, 8)
        pltpu.sync_copy(d0_hbm.at[pl.ds(base, tpw)], d0v)
        pltpu.sync_copy(d1_hbm.at[pl.ds(base, tpw)], d1v)
        pltpu.async_copy(o_hbm.at[d0v], rows, sem).wait()
        pltpu.sync_copy(rows, a_out.at[pl.ds(base, tpw)])
        pltpu.async_copy(o_hbm.at[d1v], rows, sem).wait()
        pltpu.sync_copy(rows, b_out.at[pl.ds(base, tpw)])

    return collect


_make_collect = functools.cache(_make_collect)


def _wsum_body(mf_ref, a_ref, b_ref, y_ref):
    w0 = mf_ref[:, 0:1]
    w1 = mf_ref[:, 1:2]
    y_ref[...] = w0 * a_ref[...] + w1 * b_ref[...]


def _run_wsum(meta_f, a, b):
    return pl.pallas_call(
        _wsum_body,
        grid=(8,),
        in_specs=[
            pl.BlockSpec((256, 128), lambda i: (i, 0)),
            pl.BlockSpec((256, DIM), lambda i: (i, 0)),
            pl.BlockSpec((256, DIM), lambda i: (i, 0)),
        ],
        out_specs=pl.BlockSpec((256, DIM), lambda i: (i, 0)),
        out_shape=jax.ShapeDtypeStruct((S, DIM), jnp.float32),
    )(meta_f, a, b)


def kernel(x, Wg, bg, W1, b1, W2, b2):
    xs = x.reshape(S, DIM)
    wgp = jnp.pad(Wg, ((0, 0), (0, 128 - E)))
    bgp = jnp.broadcast_to(jnp.pad(bg, (0, 128 - E))[None, :], (8, 128))

    meta_i, meta_f, off_be = _run_router(xs, wgp, bgp)
    dest_i = _run_dest(meta_i, off_be)
    dest0 = dest_i[:, 0]
    dest1 = dest_i[:, 1]

    xg = _make_dispatch()(dest0, dest1, xs)

    be = off_be[0, E:E + NB]
    o_big = _run_ffn(be, xg,
                     W1.astype(jnp.bfloat16), b1.reshape(E, 1, 2 * D_FF),
                     W2.astype(jnp.bfloat16), b2.reshape(E, 1, DIM))

    a, bb = _make_collect()(dest0, dest1, o_big)
    y = _run_wsum(meta_f, a, bb)
    return y.reshape(x.shape)


# speedup vs baseline: 1.1694x; 1.0289x over previous
import functools

import jax
import jax.numpy as jnp
from jax import lax
from jax.experimental import pallas as pl
from jax.experimental.pallas import tpu as pltpu
from jax.experimental.pallas import tpu_sc as plsc

DIM = 1024
D_FF = 1024
E = 8
TOPK = 2
S = 2048
TM = 256
NPAD = S * TOPK + E * TM
NB = NPAD // TM
NW = 32


def _router_body(x_ref, wg_ref, bg_ref, mi_ref, mf_ref, ob_ref, run_ref):
    b = pl.program_id(0)
    lane = lax.broadcasted_iota(jnp.int32, (256, 128), 1)

    @pl.when(b < 8)
    def _route():
        run = jnp.where(b == 0, 0.0, run_ref[0:1, :])
        logits = jnp.dot(x_ref[...], wg_ref[...],
                         preferred_element_type=jnp.float32) + bg_ref[0:1, :]
        masked = jnp.where(lane < E, logits, -1e30)
        v0 = jnp.max(masked, axis=-1, keepdims=True)
        e0 = jnp.min(jnp.where(masked == v0, lane, 999), axis=-1, keepdims=True)
        l1 = jnp.where(lane == e0, -1e30, masked)
        v1 = jnp.max(l1, axis=-1, keepdims=True)
        e1 = jnp.min(jnp.where(l1 == v1, lane, 999), axis=-1, keepdims=True)
        s = jnp.exp(v1 - v0)
        w0 = 1.0 / (1.0 + s)
        w1 = s / (1.0 + s)
        oh0 = (lane == e0).astype(jnp.float32)
        oh1 = (lane == e1).astype(jnp.float32)
        cnt = oh0 + oh1
        ir = lax.broadcasted_iota(jnp.int32, (256, 256), 0)
        ic = lax.broadcasted_iota(jnp.int32, (256, 256), 1)
        tri = (ir > ic).astype(jnp.float32)
        excl = jnp.dot(tri, cnt, preferred_element_type=jnp.float32) + run
        r0 = jnp.sum(excl * oh0, axis=-1, keepdims=True).astype(jnp.int32)
        r1 = jnp.sum(excl * oh1, axis=-1, keepdims=True).astype(jnp.int32)
        mi_ref[...] = jnp.where(
            lane == 0, e0, jnp.where(lane == 1, e1,
            jnp.where(lane == 2, r0, jnp.where(lane == 3, r1, 0))))
        mf_ref[...] = jnp.where(lane == 0, w0, jnp.where(lane == 1, w1, 0.0))
        run_ref[0:1, :] = run + jnp.sum(cnt, axis=0, keepdims=True)

    @pl.when(b == 8)
    def _finalize():
        lane1 = lane[0:1, :]
        tot = run_ref[0:1, :].astype(jnp.int32)
        nb = (tot + (TM - 1)) // TM
        off_vec = jnp.zeros((1, 128), jnp.int32)
        be_cnt = jnp.zeros((1, 128), jnp.int32)
        incl = jnp.int32(0)
        for e in range(E):
            nb_e = jnp.sum(jnp.where(lane1 == e, nb, 0))
            off_vec = off_vec + (incl * TM) * (lane1 == e).astype(jnp.int32)
            incl = incl + nb_e
            be_cnt = be_cnt + ((lane1 - E) >= incl).astype(jnp.int32)
        be_vec = jnp.minimum(be_cnt, E - 1)
        packed = jnp.where(lane1 < E, off_vec, be_vec)
        ob_ref[...] = jnp.broadcast_to(packed, (8, 128))


def _run_router(xs, wgp, bgp):
    return pl.pallas_call(
        _router_body,
        grid=(9,),
        in_specs=[
            pl.BlockSpec((256, DIM), lambda b: (jnp.minimum(b, 7), 0)),
            pl.BlockSpec((DIM, 128), lambda b: (0, 0)),
            pl.BlockSpec((8, 128), lambda b: (0, 0)),
        ],
        out_specs=[
            pl.BlockSpec((256, 128), lambda b: (jnp.minimum(b, 7), 0)),
            pl.BlockSpec((256, 128), lambda b: (jnp.minimum(b, 7), 0)),
            pl.BlockSpec((8, 128), lambda b: (0, 0)),
        ],
        out_shape=[
            jax.ShapeDtypeStruct((S, 128), jnp.int32),
            jax.ShapeDtypeStruct((S, 128), jnp.float32),
            jax.ShapeDtypeStruct((8, 128), jnp.int32),
        ],
        scratch_shapes=[pltpu.VMEM((8, 128), jnp.float32)],
    )(xs, wgp, bgp)


def _dest_body(mi_ref, ob_ref, d_ref):
    lane = lax.broadcasted_iota(jnp.int32, (S, 128), 1)
    off_row = ob_ref[0:1, :] * (lane[0:1, :] < E).astype(jnp.int32)
    e0 = mi_ref[:, 0:1]
    e1 = mi_ref[:, 1:2]
    d0 = mi_ref[:, 2:3] + jnp.sum(
        jnp.where(lane == e0, off_row, 0), axis=-1, keepdims=True)
    d1 = mi_ref[:, 3:4] + jnp.sum(
        jnp.where(lane == e1, off_row, 0), axis=-1, keepdims=True)
    d_ref[...] = jnp.where(lane == 0, d0, jnp.where(lane == 1, d1, 0))


def _run_dest(meta_i, off_be):
    return pl.pallas_call(
        _dest_body,
        out_shape=jax.ShapeDtypeStruct((S, 128), jnp.int32),
    )(meta_i, off_be)


def _make_dispatch():
    mesh = plsc.VectorSubcoreMesh(core_axis_name="c", subcore_axis_name="s")
    tpw = S // NW
    half = tpw // 2

    @functools.partial(
        pl.kernel, mesh=mesh,
        out_type=jax.ShapeDtypeStruct((NPAD, DIM), jnp.float32),
        scratch_types=[
            pltpu.VMEM((half,), jnp.int32),
            pltpu.VMEM((half,), jnp.int32),
            pltpu.VMEM((half,), jnp.int32),
            pltpu.VMEM((half,), jnp.int32),
            pltpu.VMEM((half, DIM), jnp.float32),
            pltpu.VMEM((half, DIM), jnp.float32),
            pltpu.SemaphoreType.DMA,
            pltpu.SemaphoreType.DMA,
        ],
    )
    def dispatch(d0_hbm, d1_hbm, x_hbm, xg_out,
                 d0a, d0b, d1a, d1b, rows0, rows1, semL, semS):
        wid = lax.axis_index("s") * 2 + lax.axis_index("c")
        base = pl.multiple_of(wid * tpw, 8)
        cpL0 = pltpu.async_copy(x_hbm.at[pl.ds(base, half)], rows0, semL)
        cpL1 = pltpu.async_copy(x_hbm.at[pl.ds(base + half, half)], rows1, semL)
        pltpu.sync_copy(d0_hbm.at[pl.ds(base, half)], d0a)
        pltpu.sync_copy(d0_hbm.at[pl.ds(base + half, half)], d0b)
        pltpu.sync_copy(d1_hbm.at[pl.ds(base, half)], d1a)
        pltpu.sync_copy(d1_hbm.at[pl.ds(base + half, half)], d1b)
        cpL0.wait()
        s00 = pltpu.async_copy(rows0, xg_out.at[d0a], semS)
        s01 = pltpu.async_copy(rows0, xg_out.at[d1a], semS)
        cpL1.wait()
        s10 = pltpu.async_copy(rows1, xg_out.at[d0b], semS)
        s11 = pltpu.async_copy(rows1, xg_out.at[d1b], semS)
        s00.wait()
        s01.wait()
        s10.wait()
        s11.wait()

    return dispatch


_make_dispatch = functools.cache(_make_dispatch)


def _ffn_body(be_ref, xg_ref, w1_ref, b1_ref, w2_ref, b2_ref, o_ref):
    xb = xg_ref[...].astype(jnp.bfloat16)
    h = jnp.dot(xb, w1_ref[0], preferred_element_type=jnp.float32) + b1_ref[0]
    a1 = h[:, :D_FF]
    a2 = h[:, D_FF:]
    act = a1 * jax.nn.sigmoid(a1) * a2
    o = jnp.dot(act.astype(jnp.bfloat16), w2_ref[0],
                preferred_element_type=jnp.float32) + b2_ref[0]
    o_ref[...] = o


def _run_ffn(be, xg, w1b, b1, w2b, b2):
    grid_spec = pltpu.PrefetchScalarGridSpec(
        num_scalar_prefetch=1,
        grid=(NB,),
        in_specs=[
            pl.BlockSpec((TM, DIM), lambda b, be: (b, 0)),
            pl.BlockSpec((1, DIM, 2 * D_FF), lambda b, be: (be[b], 0, 0)),
            pl.BlockSpec((1, 1, 2 * D_FF), lambda b, be: (be[b], 0, 0)),
            pl.BlockSpec((1, D_FF, DIM), lambda b, be: (be[b], 0, 0)),
            pl.BlockSpec((1, 1, DIM), lambda b, be: (be[b], 0, 0)),
        ],
        out_specs=pl.BlockSpec((TM, DIM), lambda b, be: (b, 0)),
    )
    return pl.pallas_call(
        _ffn_body,
        grid_spec=grid_spec,
        out_shape=jax.ShapeDtypeStruct((NPAD, DIM), jnp.float32),
    )(be, xg, w1b, b1, w2b, b2)


def _make_collect():
    mesh = plsc.VectorSubcoreMesh(core_axis_name="c", subcore_axis_name="s")
    tpw = S // NW

    @functools.partial(
        pl.kernel, mesh=mesh,
        out_type=[
            jax.ShapeDtypeStruct((S, DIM), jnp.float32),
            jax.ShapeDtypeStruct((S, DIM), jnp.float32),
        ],
        scratch_types=[
            pltpu.VMEM((tpw,), jnp.int32),
            pltpu.VMEM((tpw,), jnp.int32),
            pltpu.VMEM((16, DIM), jnp.float32),
            pltpu.VMEM((16, DIM), jnp.float32),
            pltpu.SemaphoreType.DMA,
            pltpu.SemaphoreType.DMA,
        ],
    )
    def collect(d0_hbm, d1_hbm, o_hbm, a_out, b_out,
                d0v, d1v, ra, rb, sem0, sem1):
        wid = lax.axis_index("s") * 2 + lax.axis_index("c")
        base = pl.multiple_of(wid * tpw, 8)
        pltpu.sync_copy(d0_hbm.at[pl.ds(base, tpw)], d0v)
        pltpu.sync_copy(d1_hbm.at[pl.ds(base, tpw)], d1v)

        def _round(c, _):
            sl = pl.ds(pl.multiple_of(c * 16, 16), 16)
            i0 = d0v[sl]
            i1 = d1v[sl]
            cp0 = pltpu.async_copy(o_hbm.at[i0], ra, sem0)
            cp1 = pltpu.async_copy(o_hbm.at[i1], rb, sem1)
            cp0.wait()
            cp1.wait()
            dst = pl.ds(base + c * 16, 16)
            pltpu.sync_copy(ra, a_out.at[dst])
            pltpu.sync_copy(rb, b_out.at[dst])
            return 0
        lax.fori_loop(0, tpw // 16, _round, 0)

    return collect


_make_collect = functools.cache(_make_collect)


def _wsum_body(mf_ref, a_ref, b_ref, y_ref):
    w0 = mf_ref[:, 0:1]
    w1 = mf_ref[:, 1:2]
    y_ref[...] = w0 * a_ref[...] + w1 * b_ref[...]


def _run_wsum(meta_f, a, b):
    return pl.pallas_call(
        _wsum_body,
        grid=(8,),
        in_specs=[
            pl.BlockSpec((256, 128), lambda i: (i, 0)),
            pl.BlockSpec((256, DIM), lambda i: (i, 0)),
            pl.BlockSpec((256, DIM), lambda i: (i, 0)),
        ],
        out_specs=pl.BlockSpec((256, DIM), lambda i: (i, 0)),
        out_shape=jax.ShapeDtypeStruct((S, DIM), jnp.float32),
    )(meta_f, a, b)


def kernel(x, Wg, bg, W1, b1, W2, b2):
    xs = x.reshape(S, DIM)
    wgp = jnp.pad(Wg, ((0, 0), (0, 128 - E)))
    bgp = jnp.broadcast_to(jnp.pad(bg, (0, 128 - E))[None, :], (8, 128))

    meta_i, meta_f, off_be = _run_router(xs, wgp, bgp)
    dest_i = _run_dest(meta_i, off_be)
    dest0 = dest_i[:, 0]
    dest1 = dest_i[:, 1]

    xg = _make_dispatch()(dest0, dest1, xs)

    be = off_be[0, E:E + NB]
    o_big = _run_ffn(be, xg,
                     W1.astype(jnp.bfloat16), b1.reshape(E, 1, 2 * D_FF),
                     W2.astype(jnp.bfloat16), b2.reshape(E, 1, DIM))

    a, bb = _make_collect()(dest0, dest1, o_big)
    y = _run_wsum(meta_f, a, bb)
    return y.reshape(x.shape)


# speedup vs baseline: 1.3922x; 1.1905x over previous
import functools

import jax
import jax.numpy as jnp
from jax import lax
from jax.experimental import pallas as pl
from jax.experimental.pallas import tpu as pltpu
from jax.experimental.pallas import tpu_sc as plsc

DIM = 1024
D_FF = 1024
E = 8
TOPK = 2
S = 2048
TM = 256
NPAD = S * TOPK + E * TM
NB = NPAD // TM
NW = 32


def _router_body(x_ref, wg_ref, bg_ref, mf_ref, ob_ref, d_ref,
                 run_ref, mir_ref, off_ref):
    b = pl.program_id(0)
    lane8 = lax.broadcasted_iota(jnp.int32, (256, E), 1)
    lane = lax.broadcasted_iota(jnp.int32, (256, 128), 1)

    @pl.when(b < 8)
    def _route():
        xv = x_ref[...]
        run = jnp.where(b == 0, 0.0, run_ref[0:1, :])
        logits = jnp.dot(xv, wg_ref[...],
                         preferred_element_type=jnp.float32) + bg_ref[...]
        v0 = jnp.max(logits, axis=-1, keepdims=True)
        e0 = jnp.min(jnp.where(logits == v0, lane8, 999), axis=-1,
                     keepdims=True)
        l1 = jnp.where(lane8 == e0, -1e30, logits)
        v1 = jnp.max(l1, axis=-1, keepdims=True)
        e1 = jnp.min(jnp.where(l1 == v1, lane8, 999), axis=-1, keepdims=True)
        s = jnp.exp(v1 - v0)
        w0 = 1.0 / (1.0 + s)
        w1 = s / (1.0 + s)
        oh0 = (lane == e0).astype(jnp.float32)
        oh1 = (lane == e1).astype(jnp.float32)
        cnt = oh0 + oh1
        ir = lax.broadcasted_iota(jnp.int32, (256, 256), 0)
        ic = lax.broadcasted_iota(jnp.int32, (256, 256), 1)
        tri = (ir > ic).astype(jnp.float32)
        excl = jnp.dot(tri, cnt, preferred_element_type=jnp.float32) + run
        r0 = jnp.sum(excl * oh0, axis=-1, keepdims=True).astype(jnp.int32)
        r1 = jnp.sum(excl * oh1, axis=-1, keepdims=True).astype(jnp.int32)
        mf_ref[...] = jnp.where(lane == 0, w0, jnp.where(lane == 1, w1, 0.0))
        mir_ref[pl.ds(b * 256, 256), :] = jnp.where(
            lane == 0, e0, jnp.where(lane == 1, e1,
            jnp.where(lane == 2, r0, jnp.where(lane == 3, r1, 0))))
        run_ref[0:1, :] = run + jnp.sum(cnt, axis=0, keepdims=True)

    @pl.when(b == 8)
    def _finalize():
        lane1 = lax.broadcasted_iota(jnp.int32, (1, 128), 1)
        tot = run_ref[0:1, :].astype(jnp.int32)
        nb = (tot + (TM - 1)) // TM
        off_vec = jnp.zeros((1, 128), jnp.int32)
        be_cnt = jnp.zeros((1, 128), jnp.int32)
        incl = jnp.int32(0)
        for e in range(E):
            nb_e = jnp.sum(jnp.where(lane1 == e, nb, 0))
            off_vec = off_vec + (incl * TM) * (lane1 == e).astype(jnp.int32)
            incl = incl + nb_e
            be_cnt = be_cnt + ((lane1 - E) >= incl).astype(jnp.int32)
        be_vec = jnp.minimum(be_cnt, E - 1)
        off_ref[0:1, :] = off_vec
        ob_ref[...] = jnp.broadcast_to(
            jnp.where(lane1 < E, off_vec, be_vec), (8, 128))

    @pl.when(b > 8)
    def _dest():
        i = b - 9
        mi = mir_ref[pl.ds(i * 256, 256), :]
        off_row = off_ref[0:1, :]
        e0 = mi[:, 0:1]
        e1 = mi[:, 1:2]
        d0 = mi[:, 2:3] + jnp.sum(
            jnp.where(lane == e0, off_row, 0), axis=-1, keepdims=True)
        d1 = mi[:, 3:4] + jnp.sum(
            jnp.where(lane == e1, off_row, 0), axis=-1, keepdims=True)
        d_ref[...] = jnp.where(lane == 0, d0, jnp.where(lane == 1, d1, 0))


def _run_router(xs, wg, bg1):
    def _blk(b):
        return jnp.minimum(b, 7)

    def _dblk(b):
        return jnp.maximum(b - 9, 0)

    return pl.pallas_call(
        _router_body,
        grid=(17,),
        in_specs=[
            pl.BlockSpec((256, DIM), lambda b: (_blk(b), 0)),
            pl.BlockSpec((DIM, E), lambda b: (0, 0)),
            pl.BlockSpec((1, E), lambda b: (0, 0)),
        ],
        out_specs=[
            pl.BlockSpec((256, 128), lambda b: (_blk(b), 0)),
            pl.BlockSpec((8, 128), lambda b: (0, 0)),
            pl.BlockSpec((256, 128), lambda b: (_dblk(b), 0)),
        ],
        out_shape=[
            jax.ShapeDtypeStruct((S, 128), jnp.float32),
            jax.ShapeDtypeStruct((8, 128), jnp.int32),
            jax.ShapeDtypeStruct((S, 128), jnp.int32),
        ],
        scratch_shapes=[
            pltpu.VMEM((8, 128), jnp.float32),
            pltpu.VMEM((S, 128), jnp.int32),
            pltpu.VMEM((8, 128), jnp.int32),
        ],
    )(xs, wg, bg1)


def _make_dispatch():
    mesh = plsc.VectorSubcoreMesh(core_axis_name="c", subcore_axis_name="s")
    tpw = S // NW
    half = tpw // 2

    @functools.partial(
        pl.kernel, mesh=mesh,
        out_type=jax.ShapeDtypeStruct((NPAD, DIM), jnp.float32),
        scratch_types=[
            pltpu.VMEM((half,), jnp.int32),
            pltpu.VMEM((half,), jnp.int32),
            pltpu.VMEM((half,), jnp.int32),
            pltpu.VMEM((half,), jnp.int32),
            pltpu.VMEM((half, DIM), jnp.float32),
            pltpu.VMEM((half, DIM), jnp.float32),
            pltpu.SemaphoreType.DMA,
            pltpu.SemaphoreType.DMA,
        ],
    )
    def dispatch(d0_hbm, d1_hbm, x_hbm, xg_out,
                 d0a, d0b, d1a, d1b, rows0, rows1, semL, semS):
        wid = lax.axis_index("s") * 2 + lax.axis_index("c")
        base = pl.multiple_of(wid * tpw, 8)
        cpL0 = pltpu.async_copy(x_hbm.at[pl.ds(base, half)], rows0, semL)
        cpL1 = pltpu.async_copy(x_hbm.at[pl.ds(base + half, half)], rows1, semL)
        pltpu.sync_copy(d0_hbm.at[pl.ds(base, half)], d0a)
        pltpu.sync_copy(d0_hbm.at[pl.ds(base + half, half)], d0b)
        pltpu.sync_copy(d1_hbm.at[pl.ds(base, half)], d1a)
        pltpu.sync_copy(d1_hbm.at[pl.ds(base + half, half)], d1b)
        cpL0.wait()
        s00 = pltpu.async_copy(rows0, xg_out.at[d0a], semS)
        s01 = pltpu.async_copy(rows0, xg_out.at[d1a], semS)
        cpL1.wait()
        s10 = pltpu.async_copy(rows1, xg_out.at[d0b], semS)
        s11 = pltpu.async_copy(rows1, xg_out.at[d1b], semS)
        s00.wait()
        s01.wait()
        s10.wait()
        s11.wait()

    return dispatch


_make_dispatch = functools.cache(_make_dispatch)


def _ffn_body(be_ref, xg_ref, w1_ref, b1_ref, w2_ref, b2_ref, o_ref,
              w1s, w2s):
    b = pl.program_id(0)
    e_now = be_ref[b]
    e_prev = be_ref[jnp.maximum(b - 1, 0)]

    @pl.when((b == 0) | (e_now != e_prev))
    def _convert():
        w1s[...] = w1_ref[0].astype(jnp.bfloat16)
        w2s[...] = w2_ref[0].astype(jnp.bfloat16)

    h = jnp.dot(xg_ref[...].astype(jnp.bfloat16), w1s[...],
                preferred_element_type=jnp.float32) + b1_ref[0]
    a1 = h[:, :D_FF]
    a2 = h[:, D_FF:]
    act = a1 * jax.nn.sigmoid(a1) * a2
    o = jnp.dot(act.astype(jnp.bfloat16), w2s[...],
                preferred_element_type=jnp.float32) + b2_ref[0]
    o_ref[...] = o


def _run_ffn(be, xg, w1, b1, w2, b2):
    grid_spec = pltpu.PrefetchScalarGridSpec(
        num_scalar_prefetch=1,
        grid=(NB,),
        in_specs=[
            pl.BlockSpec((TM, DIM), lambda b, be: (b, 0)),
            pl.BlockSpec((1, DIM, 2 * D_FF), lambda b, be: (be[b], 0, 0)),
            pl.BlockSpec((1, 1, 2 * D_FF), lambda b, be: (be[b], 0, 0)),
            pl.BlockSpec((1, D_FF, DIM), lambda b, be: (be[b], 0, 0)),
            pl.BlockSpec((1, 1, DIM), lambda b, be: (be[b], 0, 0)),
        ],
        out_specs=pl.BlockSpec((TM, DIM), lambda b, be: (b, 0)),
        scratch_shapes=[
            pltpu.VMEM((DIM, 2 * D_FF), jnp.bfloat16),
            pltpu.VMEM((D_FF, DIM), jnp.bfloat16),
        ],
    )
    return pl.pallas_call(
        _ffn_body,
        grid_spec=grid_spec,
        out_shape=jax.ShapeDtypeStruct((NPAD, DIM), jnp.float32),
    )(be, xg, w1, b1, w2, b2)


def _make_collect():
    mesh = plsc.VectorSubcoreMesh(core_axis_name="c", subcore_axis_name="s")
    tpw = S // NW

    @functools.partial(
        pl.kernel, mesh=mesh,
        out_type=[
            jax.ShapeDtypeStruct((S, DIM), jnp.float32),
            jax.ShapeDtypeStruct((S, DIM), jnp.float32),
        ],
        scratch_types=[
            pltpu.VMEM((tpw,), jnp.int32),
            pltpu.VMEM((tpw,), jnp.int32),
            pltpu.VMEM((16, DIM), jnp.float32),
            pltpu.VMEM((16, DIM), jnp.float32),
            pltpu.SemaphoreType.DMA,
            pltpu.SemaphoreType.DMA,
        ],
    )
    def collect(d0_hbm, d1_hbm, o_hbm, a_out, b_out,
                d0v, d1v, ra, rb, sem0, sem1):
        wid = lax.axis_index("s") * 2 + lax.axis_index("c")
        base = pl.multiple_of(wid * tpw, 8)
        pltpu.sync_copy(d0_hbm.at[pl.ds(base, tpw)], d0v)
        pltpu.sync_copy(d1_hbm.at[pl.ds(base, tpw)], d1v)

        def _round(c, _):
            sl = pl.ds(pl.multiple_of(c * 16, 16), 16)
            i0 = d0v[sl]
            i1 = d1v[sl]
            cp0 = pltpu.async_copy(o_hbm.at[i0], ra, sem0)
            cp1 = pltpu.async_copy(o_hbm.at[i1], rb, sem1)
            cp0.wait()
            cp1.wait()
            dst = pl.ds(base + c * 16, 16)
            pltpu.sync_copy(ra, a_out.at[dst])
            pltpu.sync_copy(rb, b_out.at[dst])
            return 0
        lax.fori_loop(0, tpw // 16, _round, 0)

    return collect


_make_collect = functools.cache(_make_collect)


def _wsum_body(mf_ref, a_ref, b_ref, y_ref):
    w0 = mf_ref[:, 0:1]
    w1 = mf_ref[:, 1:2]
    y_ref[...] = w0 * a_ref[...] + w1 * b_ref[...]


def _run_wsum(meta_f, a, b):
    return pl.pallas_call(
        _wsum_body,
        grid=(8,),
        in_specs=[
            pl.BlockSpec((256, 128), lambda i: (i, 0)),
            pl.BlockSpec((256, DIM), lambda i: (i, 0)),
            pl.BlockSpec((256, DIM), lambda i: (i, 0)),
        ],
        out_specs=pl.BlockSpec((256, DIM), lambda i: (i, 0)),
        out_shape=jax.ShapeDtypeStruct((S, DIM), jnp.float32),
    )(meta_f, a, b)


def kernel(x, Wg, bg, W1, b1, W2, b2):
    xs = x.reshape(S, DIM)
    meta_f, off_be, dest_i = _run_router(xs, Wg, bg.reshape(1, E))
    dest0 = dest_i[:, 0]
    dest1 = dest_i[:, 1]

    xg = _make_dispatch()(dest0, dest1, xs)

    be = off_be[0, E:E + NB]
    o_big = _run_ffn(be, xg, W1, b1.reshape(E, 1, 2 * D_FF),
                     W2, b2.reshape(E, 1, DIM))

    a, bb = _make_collect()(dest0, dest1, o_big)
    y = _run_wsum(meta_f, a, bb)
    return y.reshape(x.shape)


# speedup vs baseline: 1.6259x; 1.1679x over previous
import functools

import jax
import jax.numpy as jnp
from jax import lax
from jax.experimental import pallas as pl
from jax.experimental.pallas import tpu as pltpu
from jax.experimental.pallas import tpu_sc as plsc

DIM = 1024
D_FF = 1024
E = 8
TOPK = 2
S = 2048
TM = 256
NPAD = S * TOPK + E * TM
NB = NPAD // TM
NW = 32


RB = 512


def _router_body(x_ref, wg_ref, bg_ref, mf_ref, ob_ref, d_ref,
                 run_ref, mir_ref, off_ref):
    b = pl.program_id(0)
    lane8 = lax.broadcasted_iota(jnp.int32, (RB, E), 1)
    lane = lax.broadcasted_iota(jnp.int32, (RB, 128), 1)

    @pl.when(b < S // RB)
    def _route():
        xv = x_ref[...]
        run = jnp.where(b == 0, 0.0, run_ref[0:1, :])
        logits = jnp.dot(xv, wg_ref[...],
                         preferred_element_type=jnp.float32) + bg_ref[...]
        v0 = jnp.max(logits, axis=-1, keepdims=True)
        e0 = jnp.min(jnp.where(logits == v0, lane8, 999), axis=-1,
                     keepdims=True)
        l1 = jnp.where(lane8 == e0, -1e30, logits)
        v1 = jnp.max(l1, axis=-1, keepdims=True)
        e1 = jnp.min(jnp.where(l1 == v1, lane8, 999), axis=-1, keepdims=True)
        s = jnp.exp(v1 - v0)
        w0 = 1.0 / (1.0 + s)
        w1 = s / (1.0 + s)
        oh0 = (lane == e0).astype(jnp.float32)
        oh1 = (lane == e1).astype(jnp.float32)
        cnt = oh0 + oh1
        ir = lax.broadcasted_iota(jnp.int32, (RB, RB), 0)
        ic = lax.broadcasted_iota(jnp.int32, (RB, RB), 1)
        tri = (ir > ic).astype(jnp.float32)
        excl = jnp.dot(tri, cnt, preferred_element_type=jnp.float32) + run
        r0 = jnp.sum(excl * oh0, axis=-1, keepdims=True).astype(jnp.int32)
        r1 = jnp.sum(excl * oh1, axis=-1, keepdims=True).astype(jnp.int32)
        mf_ref[...] = jnp.where(lane == 0, w0, jnp.where(lane == 1, w1, 0.0))
        mir_ref[pl.ds(b * RB, RB), :] = jnp.where(
            lane == 0, e0, jnp.where(lane == 1, e1,
            jnp.where(lane == 2, r0, jnp.where(lane == 3, r1, 0))))
        run_ref[0:1, :] = run + jnp.sum(cnt, axis=0, keepdims=True)

    @pl.when(b == S // RB)
    def _finalize():
        lane1 = lax.broadcasted_iota(jnp.int32, (1, 128), 1)
        tot = run_ref[0:1, :].astype(jnp.int32)
        nb = (tot + (TM - 1)) // TM
        off_vec = jnp.zeros((1, 128), jnp.int32)
        be_cnt = jnp.zeros((1, 128), jnp.int32)
        ord_cnt = jnp.zeros((1, 128), jnp.int32)
        incl = jnp.int32(0)
        starts = []
        nonempties = []
        blk = lane1 - E
        for e in range(E):
            nb_e = jnp.sum(jnp.where(lane1 == e, nb, 0))
            nonempty = nb_e > 0
            starts.append(incl)
            nonempties.append(nonempty)
            off_vec = off_vec + (incl * TM) * (lane1 == e).astype(jnp.int32)
            incl = incl + nb_e
            be_cnt = be_cnt + ((lane1 - E) >= incl).astype(jnp.int32)
            ord_cnt = ord_cnt + ((blk >= starts[e]) & nonempty).astype(jnp.int32)
        be_vec = jnp.minimum(be_cnt, E - 1)
        slot_vec = (ord_cnt - 1) & 1
        nd_vec = be_vec
        for e in range(E - 1, -1, -1):
            nd_vec = jnp.where((blk < starts[e]) & nonempties[e], e, nd_vec)
        packed_be = be_vec | (slot_vec << 3) | (nd_vec << 4)
        off_ref[0:1, :] = off_vec
        packed = jnp.where(lane1 < E, off_vec,
                           jnp.where(lane1 == E + NB, incl, packed_be))
        ob_ref[...] = jnp.broadcast_to(packed, (8, 128))

    @pl.when(b > S // RB)
    def _dest():
        i = b - (S // RB + 1)
        mi = mir_ref[pl.ds(i * RB, RB), :]
        off_row = off_ref[0:1, :]
        e0 = mi[:, 0:1]
        e1 = mi[:, 1:2]
        d0 = mi[:, 2:3] + jnp.sum(
            jnp.where(lane == e0, off_row, 0), axis=-1, keepdims=True)
        d1 = mi[:, 3:4] + jnp.sum(
            jnp.where(lane == e1, off_row, 0), axis=-1, keepdims=True)
        d_ref[...] = jnp.where(lane == 0, d0, jnp.where(lane == 1, d1, 0))


def _run_router(xs, wg, bg1):
    nrb = S // RB

    def _blk(b):
        return jnp.minimum(b, nrb - 1)

    def _dblk(b):
        return jnp.maximum(b - nrb - 1, 0)

    return pl.pallas_call(
        _router_body,
        grid=(2 * nrb + 1,),
        in_specs=[
            pl.BlockSpec((RB, DIM), lambda b: (_blk(b), 0)),
            pl.BlockSpec((DIM, E), lambda b: (0, 0)),
            pl.BlockSpec((1, E), lambda b: (0, 0)),
        ],
        out_specs=[
            pl.BlockSpec((RB, 128), lambda b: (_blk(b), 0)),
            pl.BlockSpec((8, 128), lambda b: (0, 0)),
            pl.BlockSpec((RB, 128), lambda b: (_dblk(b), 0)),
        ],
        out_shape=[
            jax.ShapeDtypeStruct((S, 128), jnp.float32),
            jax.ShapeDtypeStruct((8, 128), jnp.int32),
            jax.ShapeDtypeStruct((S, 128), jnp.int32),
        ],
        scratch_shapes=[
            pltpu.VMEM((8, 128), jnp.float32),
            pltpu.VMEM((S, 128), jnp.int32),
            pltpu.VMEM((8, 128), jnp.int32),
        ],
    )(xs, wg, bg1)


def _make_dispatch():
    mesh = plsc.VectorSubcoreMesh(core_axis_name="c", subcore_axis_name="s")
    tpw = S // NW
    half = tpw // 2

    @functools.partial(
        pl.kernel, mesh=mesh,
        out_type=jax.ShapeDtypeStruct((NPAD, DIM), jnp.float32),
        scratch_types=[
            pltpu.VMEM((half,), jnp.int32),
            pltpu.VMEM((half,), jnp.int32),
            pltpu.VMEM((half,), jnp.int32),
            pltpu.VMEM((half,), jnp.int32),
            pltpu.VMEM((half, DIM), jnp.float32),
            pltpu.VMEM((half, DIM), jnp.float32),
            pltpu.SemaphoreType.DMA,
            pltpu.SemaphoreType.DMA,
        ],
    )
    def dispatch(d0_hbm, d1_hbm, x_hbm, xg_out,
                 d0a, d0b, d1a, d1b, rows0, rows1, semL, semS):
        wid = lax.axis_index("s") * 2 + lax.axis_index("c")
        base = pl.multiple_of(wid * tpw, 8)
        cpL0 = pltpu.async_copy(x_hbm.at[pl.ds(base, half)], rows0, semL)
        cpL1 = pltpu.async_copy(x_hbm.at[pl.ds(base + half, half)], rows1, semL)
        pltpu.sync_copy(d0_hbm.at[pl.ds(base, half)], d0a)
        pltpu.sync_copy(d0_hbm.at[pl.ds(base + half, half)], d0b)
        pltpu.sync_copy(d1_hbm.at[pl.ds(base, half)], d1a)
        pltpu.sync_copy(d1_hbm.at[pl.ds(base + half, half)], d1b)
        cpL0.wait()
        s00 = pltpu.async_copy(rows0, xg_out.at[d0a], semS)
        s01 = pltpu.async_copy(rows0, xg_out.at[d1a], semS)
        cpL1.wait()
        s10 = pltpu.async_copy(rows1, xg_out.at[d0b], semS)
        s11 = pltpu.async_copy(rows1, xg_out.at[d1b], semS)
        s00.wait()
        s01.wait()
        s10.wait()
        s11.wait()

    return dispatch


_make_dispatch = functools.cache(_make_dispatch)


NCH = 4
CW = D_FF // NCH


def _ffn_body(be_ref, xg_ref, w1_hbm, b1g_ref, b1u_ref, b2_ref, w2_hbm,
              o_ref, w1gs, w1us, w2s, s1, s2, sem1a, sem1b, sem2a, sem2b):
    b = pl.program_id(0)
    sems1 = (sem1a, sem1b)
    sems2 = (sem2a, sem2b)

    def _start(e_idx, sl):
        pltpu.make_async_copy(w1_hbm.at[e_idx], s1.at[sl], sems1[sl]).start()
        pltpu.make_async_copy(w2_hbm.at[e_idx], s2.at[sl], sems2[sl]).start()

    def _wait(e_idx, sl):
        pltpu.make_async_copy(w1_hbm.at[e_idx], s1.at[sl], sems1[sl]).wait()
        pltpu.make_async_copy(w2_hbm.at[e_idx], s2.at[sl], sems2[sl]).wait()

    @pl.when(b < be_ref[NB])
    def _compute():
        v = be_ref[b]
        e = v & 7
        slot = (v >> 3) & 1
        nd = (v >> 4) & 7
        v_prev = be_ref[jnp.maximum(b - 1, 0)]

        @pl.when(b == 0)
        def _prologue():
            _start(e, 0)

        @pl.when((b == 0) | (v != v_prev))
        def _turnover():
            for sl in range(2):
                @pl.when(slot == sl)
                def _sl():
                    _wait(e, sl)
                    w1gs[...] = s1[sl, :, :D_FF].astype(jnp.bfloat16)
                    w1us[...] = s1[sl, :, D_FF:].astype(jnp.bfloat16)
                    w2s[...] = s2[sl].astype(jnp.bfloat16)

                    @pl.when(nd != e)
                    def _prefetch():
                        _start(nd, 1 - sl)

        xb = xg_ref[...].astype(jnp.bfloat16)
        o_ref[...] = jnp.broadcast_to(b2_ref[0], (TM, DIM))
        for c in range(NCH):
            sl = slice(c * CW, (c + 1) * CW)
            a1 = jnp.dot(xb, w1gs[:, sl],
                         preferred_element_type=jnp.float32) + b1g_ref[0, 0:1, sl]
            a2 = jnp.dot(xb, w1us[:, sl],
                         preferred_element_type=jnp.float32) + b1u_ref[0, 0:1, sl]
            act = (a1 * jax.nn.sigmoid(a1) * a2).astype(jnp.bfloat16)
            o_ref[...] = o_ref[...] + jnp.dot(
                act, w2s[sl, :], preferred_element_type=jnp.float32)


def _run_ffn(be, xg, w1, b1v, w2, b2):
    grid_spec = pltpu.PrefetchScalarGridSpec(
        num_scalar_prefetch=1,
        grid=(NB,),
        in_specs=[
            pl.BlockSpec((TM, DIM), lambda b, be: (b, 0)),
            pl.BlockSpec(memory_space=pl.ANY),
            pl.BlockSpec((1, 1, D_FF), lambda b, be: (2 * (be[b] & 7), 0, 0)),
            pl.BlockSpec((1, 1, D_FF), lambda b, be: (2 * (be[b] & 7) + 1, 0, 0)),
            pl.BlockSpec((1, 1, DIM), lambda b, be: (be[b] & 7, 0, 0)),
            pl.BlockSpec(memory_space=pl.ANY),
        ],
        out_specs=pl.BlockSpec((TM, DIM), lambda b, be: (b, 0)),
        scratch_shapes=[
            pltpu.VMEM((DIM, D_FF), jnp.bfloat16),
            pltpu.VMEM((DIM, D_FF), jnp.bfloat16),
            pltpu.VMEM((D_FF, DIM), jnp.bfloat16),
            pltpu.VMEM((2, DIM, 2 * D_FF), jnp.float32),
            pltpu.VMEM((2, D_FF, DIM), jnp.float32),
            pltpu.SemaphoreType.DMA,
            pltpu.SemaphoreType.DMA,
            pltpu.SemaphoreType.DMA,
            pltpu.SemaphoreType.DMA,
        ],
    )
    return pl.pallas_call(
        _ffn_body,
        grid_spec=grid_spec,
        out_shape=jax.ShapeDtypeStruct((NPAD, DIM), jnp.float32),
    )(be, xg, w1, b1v, b1v, b2, w2)


def _make_collect():
    mesh = plsc.VectorSubcoreMesh(core_axis_name="c", subcore_axis_name="s")
    tpw = S // NW

    @functools.partial(
        pl.kernel, mesh=mesh,
        out_type=[
            jax.ShapeDtypeStruct((S, DIM), jnp.float32),
            jax.ShapeDtypeStruct((S, DIM), jnp.float32),
        ],
        scratch_types=[
            pltpu.VMEM((tpw,), jnp.int32),
            pltpu.VMEM((tpw,), jnp.int32),
            pltpu.VMEM((16, DIM), jnp.float32),
            pltpu.VMEM((16, DIM), jnp.float32),
            pltpu.VMEM((16, DIM), jnp.float32),
            pltpu.VMEM((16, DIM), jnp.float32),
            pltpu.SemaphoreType.DMA,
            pltpu.SemaphoreType.DMA,
            pltpu.SemaphoreType.DMA,
        ],
    )
    def collect(d0_hbm, d1_hbm, o_hbm, a_out, b_out,
                d0v, d1v, ra0, rb0, ra1, rb1, sem0, sem1, semst):
        wid = lax.axis_index("s") * 2 + lax.axis_index("c")
        base = pl.multiple_of(wid * tpw, 8)
        pltpu.sync_copy(d0_hbm.at[pl.ds(base, tpw)], d0v)
        pltpu.sync_copy(d1_hbm.at[pl.ds(base, tpw)], d1v)
        stores = []
        for c in range(tpw // 16):
            ra, rb = (ra0, rb0) if c % 2 == 0 else (ra1, rb1)
            if c >= 2:
                stores[2 * (c - 2)].wait()
                stores[2 * (c - 2) + 1].wait()
            sl = pl.ds(c * 16, 16)
            i0 = d0v[sl]
            i1 = d1v[sl]
            cp0 = pltpu.async_copy(o_hbm.at[i0], ra, sem0)
            cp1 = pltpu.async_copy(o_hbm.at[i1], rb, sem1)
            cp0.wait()
            cp1.wait()
            dst = pl.ds(base + c * 16, 16)
            stores.append(pltpu.async_copy(ra, a_out.at[dst], semst))
            stores.append(pltpu.async_copy(rb, b_out.at[dst], semst))
        for st in stores[-4:]:
            st.wait()

    return collect


_make_collect = functools.cache(_make_collect)


def _wsum_body(mf_ref, a_ref, b_ref, y_ref):
    w0 = mf_ref[:, 0:1]
    w1 = mf_ref[:, 1:2]
    y_ref[...] = w0 * a_ref[...] + w1 * b_ref[...]


def _run_wsum(meta_f, a, b):
    return pl.pallas_call(
        _wsum_body,
        grid=(4,),
        in_specs=[
            pl.BlockSpec((512, 128), lambda i: (i, 0)),
            pl.BlockSpec((512, DIM), lambda i: (i, 0)),
            pl.BlockSpec((512, DIM), lambda i: (i, 0)),
        ],
        out_specs=pl.BlockSpec((512, DIM), lambda i: (i, 0)),
        out_shape=jax.ShapeDtypeStruct((S, DIM), jnp.float32),
    )(meta_f, a, b)


def kernel(x, Wg, bg, W1, b1, W2, b2):
    xs = x.reshape(S, DIM)
    meta_f, off_be, dest_i = _run_router(xs, Wg, bg.reshape(1, E))
    dest0 = dest_i[:, 0]
    dest1 = dest_i[:, 1]

    xg = _make_dispatch()(dest0, dest1, xs)

    be = off_be[0, E:E + NB + 1]
    o_big = _run_ffn(be, xg, W1, b1.reshape(2 * E, 1, D_FF),
                     W2, b2.reshape(E, 1, DIM))

    a, bb = _make_collect()(dest0, dest1, o_big)
    y = _run_wsum(meta_f, a, bb)
    return y.reshape(x.shape)
